# baseline (speedup 1.0000x reference)
import functools

import jax
import jax.numpy as jnp
from jax import lax
from jax.experimental import pallas as pl
from jax.experimental.pallas import tpu as pltpu
from jax.experimental.pallas import tpu_sc as plsc

VOCAB = 1000000
TEXT_DIM = 64
ENC_DIM = 128
BATCH = 4096
SEQ = 200

NT = BATCH * SEQ
NC = 2
NS = 16
NW = NC * NS
B_PER_W = NT // NW
CHUNK = 128
NCHUNK = B_PER_W // CHUNK


def _gather_body(idx_hbm, table_hbm, out_hbm, idx_v, rows_v, sem):
    wid = lax.axis_index("s") * NC + lax.axis_index("c")
    base = wid * B_PER_W
    pltpu.sync_copy(idx_hbm.at[wid], idx_v)

    def body(j, carry):
        pltpu.async_copy(table_hbm.at[idx_v.at[j]], rows_v, sem).wait()
        pltpu.sync_copy(rows_v, out_hbm.at[pl.ds(base + j * CHUNK, CHUNK)])
        return carry

    lax.fori_loop(0, NCHUNK, body, 0)


_gather = functools.partial(
    pl.kernel,
    mesh=plsc.VectorSubcoreMesh(core_axis_name="c", subcore_axis_name="s"),
    compiler_params=pltpu.CompilerParams(use_tc_tiling_on_sc=False),
    out_type=jax.ShapeDtypeStruct((NT, TEXT_DIM), jnp.float32),
    scratch_types=[
        pltpu.VMEM((NCHUNK, CHUNK), jnp.int32),
        pltpu.VMEM((CHUNK, TEXT_DIM), jnp.float32),
        pltpu.SemaphoreType.DMA,
    ],
)(_gather_body)


TOK_BLK = 4096


def _mm_body(emb_ref, w_ref, b_ref, out_ref):
    out_ref[...] = (
        jnp.dot(emb_ref[...], w_ref[...], preferred_element_type=jnp.float32)
        + b_ref[...]
    )


_mm = pl.pallas_call(
    _mm_body,
    grid=(NT // TOK_BLK,),
    in_specs=[
        pl.BlockSpec((TOK_BLK, TEXT_DIM), lambda i: (i, 0)),
        pl.BlockSpec((TEXT_DIM, ENC_DIM), lambda i: (0, 0)),
        pl.BlockSpec((1, ENC_DIM), lambda i: (0, 0)),
    ],
    out_specs=pl.BlockSpec((TOK_BLK, ENC_DIM), lambda i: (i, 0)),
    out_shape=jax.ShapeDtypeStruct((NT, ENC_DIM), jnp.float32),
)


@jax.jit
def kernel(x, table, W, b):
    idx = x.reshape(NW, NCHUNK, CHUNK)
    emb = _gather(idx, table)
    out = _mm(emb, W, b.reshape(1, ENC_DIM))
    return out.reshape(BATCH, SEQ, ENC_DIM)


# speedup vs baseline: 2.0475x; 2.0475x over previous
import functools

import jax
import jax.numpy as jnp
from jax import lax
from jax.experimental import pallas as pl
from jax.experimental.pallas import tpu as pltpu
from jax.experimental.pallas import tpu_sc as plsc

VOCAB = 1000000
TEXT_DIM = 64
ENC_DIM = 128
BATCH = 4096
SEQ = 200

NT = BATCH * SEQ
NC = 2
NS = 16
NW = NC * NS
B_PER_W = NT // NW
CHUNK = 128
NCHUNK = B_PER_W // CHUNK

V_BLK = 8192


def _proj_body(tT_ref, w_ref, b_ref, out_ref):
    out_ref[...] = (
        lax.dot_general(
            tT_ref[...], w_ref[...],
            dimension_numbers=(((0,), (0,)), ((), ())),
            preferred_element_type=jnp.float32,
        )
        + b_ref[...]
    )


_project = pl.pallas_call(
    _proj_body,
    grid=(pl.cdiv(VOCAB, V_BLK),),
    in_specs=[
        pl.BlockSpec((TEXT_DIM, V_BLK), lambda i: (0, i)),
        pl.BlockSpec((TEXT_DIM, ENC_DIM), lambda i: (0, 0)),
        pl.BlockSpec((1, ENC_DIM), lambda i: (0, 0)),
    ],
    out_specs=pl.BlockSpec((V_BLK, ENC_DIM), lambda i: (i, 0)),
    out_shape=jax.ShapeDtypeStruct((VOCAB, ENC_DIM), jnp.float32),
)


def _gather_body(idx_hbm, pt_hbm, out_hbm, idx_v, rows_a, rows_b, sem):
    wid = lax.axis_index("s") * NC + lax.axis_index("c")
    base = wid * B_PER_W
    pltpu.sync_copy(idx_hbm.at[wid], idx_v)

    def body(j, carry):
        pltpu.async_copy(pt_hbm.at[idx_v.at[j]], rows_a, sem).wait()
        pltpu.sync_copy(rows_a, out_hbm.at[pl.ds(base + j * CHUNK, CHUNK)])
        return carry

    lax.fori_loop(0, NCHUNK, body, 0)


_gather = functools.partial(
    pl.kernel,
    mesh=plsc.VectorSubcoreMesh(core_axis_name="c", subcore_axis_name="s"),
    compiler_params=pltpu.CompilerParams(use_tc_tiling_on_sc=False),
    out_type=jax.ShapeDtypeStruct((NT, ENC_DIM), jnp.float32),
    scratch_types=[
        pltpu.VMEM((NCHUNK, CHUNK), jnp.int32),
        pltpu.VMEM((CHUNK, ENC_DIM), jnp.float32),
        pltpu.VMEM((CHUNK, ENC_DIM), jnp.float32),
        pltpu.SemaphoreType.DMA,
    ],
)(_gather_body)


@jax.jit
def kernel(x, table, W, b):
    pt = _project(table.T, W, b.reshape(1, ENC_DIM))
    idx = x.reshape(NW, NCHUNK, CHUNK)
    out = _gather(idx, pt)
    return out.reshape(BATCH, SEQ, ENC_DIM)


# speedup vs baseline: 2.5699x; 1.2551x over previous
import functools

import jax
import jax.numpy as jnp
from jax import lax
from jax.experimental import pallas as pl
from jax.experimental.pallas import tpu as pltpu
from jax.experimental.pallas import tpu_sc as plsc

VOCAB = 1000000
TEXT_DIM = 64
ENC_DIM = 128
BATCH = 4096
SEQ = 200

NT = BATCH * SEQ
NC = 2
NS = 16
NW = NC * NS
B_PER_W = NT // NW
CHUNK = 128
NCHUNK = B_PER_W // CHUNK

V_BLK = 8192


def _proj_body(tT_ref, w_ref, b_ref, out_ref):
    out_ref[...] = (
        lax.dot_general(
            tT_ref[...], w_ref[...],
            dimension_numbers=(((0,), (0,)), ((), ())),
            preferred_element_type=jnp.float32,
        )
        + b_ref[...]
    )


_project = pl.pallas_call(
    _proj_body,
    grid=(pl.cdiv(VOCAB, V_BLK),),
    in_specs=[
        pl.BlockSpec((TEXT_DIM, V_BLK), lambda i: (0, i)),
        pl.BlockSpec((TEXT_DIM, ENC_DIM), lambda i: (0, 0)),
        pl.BlockSpec((1, ENC_DIM), lambda i: (0, 0)),
    ],
    out_specs=pl.BlockSpec((V_BLK, ENC_DIM), lambda i: (i, 0)),
    out_shape=jax.ShapeDtypeStruct((VOCAB, ENC_DIM), jnp.float32),
)


def _gather_body(idx_hbm, pt_hbm, out_hbm, idx_v, rows_a, rows_b, sem):
    wid = lax.axis_index("s") * NC + lax.axis_index("c")
    base = wid * B_PER_W
    pltpu.sync_copy(idx_hbm.at[wid], idx_v)

    pltpu.async_copy(pt_hbm.at[idx_v.at[0]], rows_a, sem)

    def body(g, carry):
        j0 = g * 2
        pltpu.async_copy(pt_hbm.at[idx_v.at[j0 + 1]], rows_b, sem)
        pltpu.make_async_copy(pt_hbm.at[idx_v.at[j0]], rows_a, sem).wait()
        pltpu.sync_copy(rows_a, out_hbm.at[pl.ds(base + j0 * CHUNK, CHUNK)])

        @pl.when(j0 + 2 < NCHUNK)
        def _():
            pltpu.async_copy(pt_hbm.at[idx_v.at[j0 + 2]], rows_a, sem)

        pltpu.make_async_copy(pt_hbm.at[idx_v.at[j0 + 1]], rows_b, sem).wait()
        pltpu.sync_copy(rows_b, out_hbm.at[pl.ds(base + (j0 + 1) * CHUNK, CHUNK)])
        return carry

    lax.fori_loop(0, NCHUNK // 2, body, 0)


_gather = functools.partial(
    pl.kernel,
    mesh=plsc.VectorSubcoreMesh(core_axis_name="c", subcore_axis_name="s"),
    compiler_params=pltpu.CompilerParams(use_tc_tiling_on_sc=False),
    out_type=jax.ShapeDtypeStruct((NT, ENC_DIM), jnp.float32),
    scratch_types=[
        pltpu.VMEM((NCHUNK, CHUNK), jnp.int32),
        pltpu.VMEM((CHUNK, ENC_DIM), jnp.float32),
        pltpu.VMEM((CHUNK, ENC_DIM), jnp.float32),
        pltpu.SemaphoreType.DMA,
    ],
)(_gather_body)


@jax.jit
def kernel(x, table, W, b):
    pt = _project(table.T, W, b.reshape(1, ENC_DIM))
    idx = x.reshape(NW, NCHUNK, CHUNK)
    out = _gather(idx, pt)
    return out.reshape(BATCH, SEQ, ENC_DIM)
